# Initial kernel scaffold; baseline (speedup 1.0000x reference)
import jax
import jax.numpy as jnp
from jax.experimental import pallas as pl


def kernel(x, edge_index, edge_attr, batch, W_rel0, b_rel0, W_root0, W_rel1, b_rel1, W_root1, W_rel2, b_rel2, W_root2):
    raise NotImplementedError("write your pallas kernel here")


# speedup vs baseline: 4.1118x; 4.1118x over previous
import functools

import jax
import jax.numpy as jnp
from jax import lax
from jax.experimental import pallas as pl
from jax.experimental.pallas import tpu as pltpu
from jax.experimental.pallas import tpu_sc as plsc

N = 10000
E = 320000
D = 128

NC = 2
NS = 16
L = 16

EDGES_PER_CORE = E // NC
EDGES_PER_TILE = E // (NC * NS)
CHUNK = 80
NCHUNKS = EDGES_PER_TILE // CHUNK
ROWS_PER_TILE = 624

_TC_BLK = 1000


def _tc_pre_body(h_ref, wr_ref, wt_ref, b_ref, y_ref, z_ref):
    h = h_ref[...]
    y_ref[...] = jnp.dot(h, wr_ref[...], preferred_element_type=jnp.float32)
    z_ref[...] = jnp.dot(h, wt_ref[...], preferred_element_type=jnp.float32) + b_ref[...]


def _tc_pre(h, wr, wt, b):
    grid = (N // _TC_BLK,)
    return pl.pallas_call(
        _tc_pre_body,
        grid=grid,
        in_specs=[
            pl.BlockSpec((_TC_BLK, D), lambda i: (i, 0)),
            pl.BlockSpec((D, D), lambda i: (0, 0)),
            pl.BlockSpec((D, D), lambda i: (0, 0)),
            pl.BlockSpec((1, D), lambda i: (0, 0)),
        ],
        out_specs=[
            pl.BlockSpec((_TC_BLK, D), lambda i: (i, 0)),
            pl.BlockSpec((_TC_BLK, D), lambda i: (i, 0)),
        ],
        out_shape=[
            jax.ShapeDtypeStruct((N, D), jnp.float32),
            jax.ShapeDtypeStruct((N, D), jnp.float32),
        ],
    )(h, wr, wt, b.reshape(1, D))


def _tc_mid_body(p_ref, z_ref, wr_ref, wt_ref, b_ref, y_ref, z2_ref):
    h = jax.nn.relu(p_ref[0] + p_ref[1] + z_ref[...])
    y_ref[...] = jnp.dot(h, wr_ref[...], preferred_element_type=jnp.float32)
    z2_ref[...] = jnp.dot(h, wt_ref[...], preferred_element_type=jnp.float32) + b_ref[...]


def _tc_mid(p, z, wr, wt, b):
    grid = (N // _TC_BLK,)
    return pl.pallas_call(
        _tc_mid_body,
        grid=grid,
        in_specs=[
            pl.BlockSpec((2, _TC_BLK, D), lambda i: (0, i, 0)),
            pl.BlockSpec((_TC_BLK, D), lambda i: (i, 0)),
            pl.BlockSpec((D, D), lambda i: (0, 0)),
            pl.BlockSpec((D, D), lambda i: (0, 0)),
            pl.BlockSpec((1, D), lambda i: (0, 0)),
        ],
        out_specs=[
            pl.BlockSpec((_TC_BLK, D), lambda i: (i, 0)),
            pl.BlockSpec((_TC_BLK, D), lambda i: (i, 0)),
        ],
        out_shape=[
            jax.ShapeDtypeStruct((N, D), jnp.float32),
            jax.ShapeDtypeStruct((N, D), jnp.float32),
        ],
    )(p, z, wr, wt, b.reshape(1, D))


def _tc_post_body(p_ref, z_ref, o_ref):
    o_ref[...] = jax.nn.relu(p_ref[0] + p_ref[1] + z_ref[...])


def _tc_post(p, z):
    grid = (N // _TC_BLK,)
    return pl.pallas_call(
        _tc_post_body,
        grid=grid,
        in_specs=[
            pl.BlockSpec((2, _TC_BLK, D), lambda i: (0, i, 0)),
            pl.BlockSpec((_TC_BLK, D), lambda i: (i, 0)),
        ],
        out_specs=pl.BlockSpec((_TC_BLK, D), lambda i: (i, 0)),
        out_shape=jax.ShapeDtypeStruct((N, D), jnp.float32),
    )(p, z)


def _sc_agg_body(y_hbm, src_hbm, dst_hbm, w_hbm, out_hbm,
                 src_v, dst_v, w_v, rows_v, zbuf_v, acc_sh, sem):
    c = lax.axis_index("c")
    s = lax.axis_index("s")

    def zb_body(i, _):
        for g in range(D // L):
            zbuf_v[i, pl.ds(g * L, L)] = jnp.zeros((L,), jnp.float32)
        return 0
    lax.fori_loop(0, CHUNK, zb_body, 0)

    r0 = s * ROWS_PER_TILE
    for k in range(7):
        pltpu.sync_copy(zbuf_v, acc_sh.at[pl.ds(r0 + k * CHUNK, CHUNK)])

    @pl.when(s == NS - 1)
    def _():
        pltpu.sync_copy(zbuf_v, acc_sh.at[pl.ds(r0 + 560, CHUNK)])

    @pl.when(s < NS - 1)
    def _():
        pltpu.sync_copy(zbuf_v.at[pl.ds(0, 64)], acc_sh.at[pl.ds(r0 + 560, 64)])

    plsc.subcore_barrier()

    ebase = c * EDGES_PER_CORE + s * EDGES_PER_TILE

    def chunk_body(k, _):
        b = ebase + k * CHUNK
        pltpu.sync_copy(src_hbm.at[pl.ds(b, CHUNK)], src_v)
        pltpu.sync_copy(dst_hbm.at[pl.ds(b, CHUNK)], dst_v)
        pltpu.sync_copy(w_hbm.at[pl.ds(b, CHUNK)], w_v)
        pltpu.async_copy(y_hbm.at[src_v], rows_v, sem).wait()

        def grp_body(gi, _):
            w16 = w_v[pl.ds(gi * L, L)]
            for i in range(L):
                wb = jnp.full((L,), w16[i], dtype=jnp.float32)
                r = gi * L + i
                for g in range(D // L):
                    rows_v[r, pl.ds(g * L, L)] = rows_v[r, pl.ds(g * L, L)] * wb
            return 0
        lax.fori_loop(0, CHUNK // L, grp_body, 0)

        pltpu.sync_copy(rows_v, acc_sh.at[dst_v], add=True)
        return 0

    lax.fori_loop(0, NCHUNKS, chunk_body, 0)

    plsc.subcore_barrier()

    ob = c * N + r0

    @pl.when(s == NS - 1)
    def _():
        pltpu.sync_copy(acc_sh.at[pl.ds(r0, 640)], out_hbm.at[pl.ds(ob, 640)])

    @pl.when(s < NS - 1)
    def _():
        pltpu.sync_copy(acc_sh.at[pl.ds(r0, ROWS_PER_TILE)],
                        out_hbm.at[pl.ds(ob, ROWS_PER_TILE)])


@functools.partial(jax.jit, static_argnames=())
def _sc_agg(y, src, dst, w):
    mesh = plsc.VectorSubcoreMesh(core_axis_name="c", subcore_axis_name="s",
                                  num_cores=NC, num_subcores=NS)
    k = pl.kernel(
        _sc_agg_body,
        out_type=jax.ShapeDtypeStruct((2 * N, D), jnp.float32),
        mesh=mesh,
        scratch_types=[
            pltpu.VMEM((CHUNK,), jnp.int32),
            pltpu.VMEM((CHUNK,), jnp.int32),
            pltpu.VMEM((CHUNK,), jnp.float32),
            pltpu.VMEM((CHUNK, D), jnp.float32),
            pltpu.VMEM((CHUNK, D), jnp.float32),
            pltpu.VMEM_SHARED((N, D), jnp.float32),
            pltpu.SemaphoreType.DMA,
        ],
    )
    return k(y, src, dst, w).reshape(2, N, D)


def kernel(x, edge_index, edge_attr, batch,
           W_rel0, b_rel0, W_root0,
           W_rel1, b_rel1, W_root1,
           W_rel2, b_rel2, W_root2):
    src = edge_index[0]
    dst = edge_index[1]

    y0, z0 = _tc_pre(x, W_rel0, W_root0, b_rel0)
    p0 = _sc_agg(y0, src, dst, edge_attr)
    y1, z1 = _tc_mid(p0, z0, W_rel1, W_root1, b_rel1)
    p1 = _sc_agg(y1, src, dst, edge_attr)
    y2, z2 = _tc_mid(p1, z1, W_rel2, W_root2, b_rel2)
    p2 = _sc_agg(y2, src, dst, edge_attr)
    return _tc_post(p2, z2)
